# baseline (speedup 1.0000x reference)
import functools

import jax
import jax.numpy as jnp
from jax import lax
from jax.experimental import pallas as pl
from jax.experimental.pallas import tpu as pltpu
from jax.experimental.pallas import tpu_sc as plsc

_LANES = 16
_CHUNK = 128


def _embed_kernel(n_flat, d, n_workers, n_chunks, scale):
    mesh = plsc.VectorSubcoreMesh(core_axis_name="c", subcore_axis_name="s")
    per_w = n_flat // n_workers

    @functools.partial(
        pl.kernel,
        mesh=mesh,
        out_type=jax.ShapeDtypeStruct((n_flat, d), jnp.float32),
        scratch_types=[
            pltpu.VMEM((n_chunks, _CHUNK), jnp.int32),
            pltpu.VMEM((_CHUNK, d), jnp.float32),
            pltpu.SemaphoreType.DMA,
        ],
        compiler_params=pltpu.CompilerParams(use_tc_tiling_on_sc=False),
    )
    def run(table_hbm, idx_hbm, out_hbm, idx_v, gbuf, sem):
        cid = lax.axis_index("c")
        sid = lax.axis_index("s")
        wid = sid * 2 + cid
        base = wid * per_w
        pltpu.sync_copy(idx_hbm.at[wid], idx_v)

        def chunk_body(j, carry):
            pltpu.async_copy(table_hbm.at[idx_v.at[j]], gbuf, sem).wait()

            def row_body(i, c):
                for k in range(d // _LANES):
                    sl = pl.ds(k * _LANES, _LANES)
                    gbuf[i, sl] = gbuf[i, sl] * scale
                return c

            lax.fori_loop(0, _CHUNK, row_body, 0)
            pltpu.sync_copy(gbuf, out_hbm.at[pl.ds(base + j * _CHUNK, _CHUNK)])
            return carry

        lax.fori_loop(0, n_chunks, chunk_body, 0)

    return run


def kernel(table, x):
    v, d = table.shape
    b, l = x.shape
    n_flat = b * l
    n_workers = 32
    per_w = n_flat // n_workers
    n_chunks = per_w // _CHUNK
    scale = float(d) ** -0.5

    idx3 = x.reshape(n_workers, n_chunks, _CHUNK)
    out = _embed_kernel(n_flat, d, n_workers, n_chunks, scale)(table, idx3)
    return out.reshape(b, l, d)


# speedup vs baseline: 1.1856x; 1.1856x over previous
import functools

import jax
import jax.numpy as jnp
from jax import lax
from jax.experimental import pallas as pl
from jax.experimental.pallas import tpu as pltpu
from jax.experimental.pallas import tpu_sc as plsc

_LANES = 16
_CHUNK = 128
_NBUF = 2


def _embed_kernel(n_flat, d, n_workers, n_chunks, scale):
    mesh = plsc.VectorSubcoreMesh(core_axis_name="c", subcore_axis_name="s")
    per_w = n_flat // n_workers
    kvecs = d // _LANES

    @functools.partial(
        pl.kernel,
        mesh=mesh,
        out_type=jax.ShapeDtypeStruct((n_flat, d), jnp.float32),
        scratch_types=[
            pltpu.VMEM((n_chunks, _CHUNK), jnp.int32),
            [pltpu.VMEM((_CHUNK, d), jnp.float32)] * _NBUF,
            [pltpu.VMEM((_CHUNK, d), jnp.float32)] * _NBUF,
            [pltpu.SemaphoreType.DMA] * _NBUF,
            [pltpu.SemaphoreType.DMA] * _NBUF,
        ],
        compiler_params=pltpu.CompilerParams(use_tc_tiling_on_sc=False),
    )
    def run(table_hbm, idx_hbm, out_hbm, idx_v, gb, ob, sg, so):
        cid = lax.axis_index("c")
        sid = lax.axis_index("s")
        wid = sid * 2 + cid
        base = wid * per_w
        pltpu.sync_copy(idx_hbm.at[wid], idx_v)

        for b in range(_NBUF):
            pltpu.async_copy(table_hbm.at[idx_v.at[b]], gb[b], sg[b])

        def group_body(p, carry):
            for b in range(_NBUF):
                j = p * _NBUF + b
                pltpu.make_async_copy(
                    out_hbm.at[pl.ds(base, _CHUNK)], gb[b], sg[b]
                ).wait()

                @pl.when(p > 0)
                def _():
                    pltpu.make_async_copy(
                        ob[b], out_hbm.at[pl.ds(base, _CHUNK)], so[b]
                    ).wait()

                @plsc.parallel_loop(0, _CHUNK, unroll=4)
                def _(i):
                    for k in range(kvecs):
                        sl = pl.ds(k * _LANES, _LANES)
                        ob[b][i, sl] = gb[b][i, sl] * scale

                pltpu.async_copy(
                    ob[b], out_hbm.at[pl.ds(base + j * _CHUNK, _CHUNK)], so[b]
                )

                @pl.when(j + _NBUF < n_chunks)
                def _():
                    pltpu.async_copy(
                        table_hbm.at[idx_v.at[j + _NBUF]], gb[b], sg[b]
                    )

            return carry

        lax.fori_loop(0, n_chunks // _NBUF, group_body, 0)

        for b in range(_NBUF):
            pltpu.make_async_copy(
                ob[b], out_hbm.at[pl.ds(base, _CHUNK)], so[b]
            ).wait()

    return run


def kernel(table, x):
    v, d = table.shape
    b, l = x.shape
    n_flat = b * l
    n_workers = 32
    per_w = n_flat // n_workers
    n_chunks = per_w // _CHUNK
    scale = float(d) ** -0.5

    idx3 = x.reshape(n_workers, n_chunks, _CHUNK)
    out = _embed_kernel(n_flat, d, n_workers, n_chunks, scale)(table, idx3)
    return out.reshape(b, l, d)


# speedup vs baseline: 1.1858x; 1.0002x over previous
import functools

import jax
import jax.numpy as jnp
from jax import lax
from jax.experimental import pallas as pl
from jax.experimental.pallas import tpu as pltpu
from jax.experimental.pallas import tpu_sc as plsc

_LANES = 16
_BI = 128
_CI = 8
_NBUF = 2


def _embed_kernel(seq_len, d, n_jb, scale):
    mesh = plsc.VectorSubcoreMesh(core_axis_name="c", subcore_axis_name="s")
    n_jc = d // _CI

    @functools.partial(
        pl.kernel,
        mesh=mesh,
        out_type=jax.ShapeDtypeStruct((seq_len, n_jc, n_jb, _CI, _BI), jnp.float32),
        scratch_types=[
            pltpu.VMEM((seq_len, _BI), jnp.int32),
            [pltpu.VMEM((_BI, d), jnp.float32)] * _NBUF,
            [pltpu.VMEM((n_jc, _CI, _BI), jnp.float32)] * _NBUF,
            [pltpu.SemaphoreType.DMA] * _NBUF,
            [pltpu.SemaphoreType.DMA] * _NBUF,
        ],
        compiler_params=pltpu.CompilerParams(
            use_tc_tiling_on_sc=False, needs_layout_passes=False
        ),
    )
    def run(table_hbm, idx_hbm, out_hbm, idx_v, gb, ob, sg, so):
        cid = lax.axis_index("c")
        sid = lax.axis_index("s")
        w = sid * 2 + cid
        pltpu.sync_copy(idx_hbm.at[:, w], idx_v)

        iota = jnp.arange(_LANES, dtype=jnp.int32)

        for b in range(_NBUF):
            pltpu.async_copy(table_hbm.at[idx_v.at[b]], gb[b], sg[b])

        def group_body(p, carry):
            for b in range(_NBUF):
                j = p * _NBUF + b
                pltpu.make_async_copy(
                    table_hbm.at[pl.ds(0, _BI)], gb[b], sg[b]
                ).wait()

                @pl.when(p > 0)
                def _():
                    pltpu.make_async_copy(
                        ob[b], out_hbm.at[0, :, w], so[b]
                    ).wait()

                @plsc.parallel_loop(0, d, unroll=2)
                def _(c):
                    jc = c // _CI
                    ci = c % _CI
                    col = jnp.broadcast_to(c, (_LANES,))
                    for b0 in range(_BI // _LANES):
                        rows = iota + (b0 * _LANES)
                        v = plsc.load_gather(gb[b], [rows, col])
                        ob[b][jc, ci, pl.ds(b0 * _LANES, _LANES)] = v * scale

                pltpu.async_copy(ob[b], out_hbm.at[j, :, w], so[b])

                @pl.when(j + _NBUF < seq_len)
                def _():
                    pltpu.async_copy(
                        table_hbm.at[idx_v.at[j + _NBUF]], gb[b], sg[b]
                    )

            return carry

        lax.fori_loop(0, seq_len // _NBUF, group_body, 0)

        for b in range(_NBUF):
            pltpu.make_async_copy(ob[b], out_hbm.at[0, :, w], so[b]).wait()

    return run


def kernel(table, x):
    v, d = table.shape
    bsz, seq_len = x.shape
    n_jb = bsz // _BI
    scale = float(d) ** -0.5

    idx = x.T.reshape(seq_len, n_jb, _BI)
    out5 = _embed_kernel(seq_len, d, n_jb, scale)(table, idx)
    out = out5.transpose(2, 4, 0, 1, 3).reshape(bsz, seq_len, d)
    return out


# speedup vs baseline: 1.8787x; 1.5843x over previous
import functools

import jax
import jax.numpy as jnp
from jax import lax
from jax.experimental import pallas as pl
from jax.experimental.pallas import tpu as pltpu
from jax.experimental.pallas import tpu_sc as plsc

_LANES = 16
_BI = 128
_CI = 8
_NBUF = 2


def _embed_kernel(seq_len, d, n_jb, scale):
    mesh = plsc.VectorSubcoreMesh(core_axis_name="c", subcore_axis_name="s")
    n_jc = d // _CI

    @functools.partial(
        pl.kernel,
        mesh=mesh,
        out_type=jax.ShapeDtypeStruct((seq_len, n_jc, n_jb, _CI, _BI), jnp.float32),
        scratch_types=[
            pltpu.VMEM((seq_len, _BI), jnp.int32),
            [pltpu.VMEM((_BI, d), jnp.float32)] * _NBUF,
            [pltpu.VMEM((d, _BI + 1), jnp.float32)] * _NBUF,
            [pltpu.SemaphoreType.DMA] * _NBUF,
            [pltpu.SemaphoreType.DMA] * _NBUF,
        ],
        compiler_params=pltpu.CompilerParams(
            use_tc_tiling_on_sc=False, needs_layout_passes=False
        ),
    )
    def run(table_hbm, idx_hbm, out_hbm, idx_v, gb, ob, sg, so):
        cid = lax.axis_index("c")
        sid = lax.axis_index("s")
        w = sid * 2 + cid
        pltpu.sync_copy(idx_hbm.at[:, w], idx_v)

        iota = jnp.arange(_LANES, dtype=jnp.int32)

        for b in range(_NBUF):
            pltpu.async_copy(table_hbm.at[idx_v.at[b]], gb[b], sg[b])

        def group_body(p, carry):
            for b in range(_NBUF):
                j = p * _NBUF + b
                pltpu.make_async_copy(
                    table_hbm.at[pl.ds(0, _BI)], gb[b], sg[b]
                ).wait()

                @pl.when(p > 0)
                def _():
                    for jc in range(n_jc):
                        pltpu.make_async_copy(
                            ob[b].at[pl.ds(jc * _CI, _CI), pl.ds(0, _BI)],
                            out_hbm.at[0, jc, w],
                            so[b],
                        ).wait()

                @plsc.parallel_loop(0, _BI, unroll=2)
                def _(t):
                    t_vec = jnp.broadcast_to(t, (_LANES,))
                    for k in range(d // _LANES):
                        c_idx = iota + (k * _LANES)
                        v = gb[b][t, pl.ds(k * _LANES, _LANES)] * scale
                        plsc.store_scatter(ob[b], [c_idx, t_vec], v)

                for jc in range(n_jc):
                    pltpu.async_copy(
                        ob[b].at[pl.ds(jc * _CI, _CI), pl.ds(0, _BI)],
                        out_hbm.at[j, jc, w],
                        so[b],
                    )

                @pl.when(j + _NBUF < seq_len)
                def _():
                    pltpu.async_copy(
                        table_hbm.at[idx_v.at[j + _NBUF]], gb[b], sg[b]
                    )

            return carry

        lax.fori_loop(0, seq_len // _NBUF, group_body, 0)

        for b in range(_NBUF):
            for jc in range(n_jc):
                pltpu.make_async_copy(
                    ob[b].at[pl.ds(jc * _CI, _CI), pl.ds(0, _BI)],
                    out_hbm.at[0, jc, w],
                    so[b],
                ).wait()

    return run


def kernel(table, x):
    v, d = table.shape
    bsz, seq_len = x.shape
    n_jb = bsz // _BI
    scale = float(d) ** -0.5

    idx = x.T.reshape(seq_len, n_jb, _BI)
    out5 = _embed_kernel(seq_len, d, n_jb, scale)(table, idx)
    out = out5.transpose(2, 4, 0, 1, 3).reshape(bsz, seq_len, d)
    return out
